# baseline (speedup 1.0000x reference)
import functools

import jax
import jax.numpy as jnp
from jax import lax
from jax.experimental import pallas as pl
from jax.experimental.pallas import tpu as pltpu
from jax.experimental.pallas import tpu_sc as plsc

VOCAB = 100000
DIM = 64
BATCH = 1024
CTX = 20

NUM_CORES = 2
NUM_SUBCORES = 16
LANES = 16
NW = NUM_CORES * NUM_SUBCORES
BPW = BATCH // NW
TPW = BPW * CTX
HP = 128


def _pool_body(x_hbm, emb_hbm, h_hbm, idx_v, tiles_v, h_v, sem):
    wid = lax.axis_index("s") * NUM_CORES + lax.axis_index("c")
    pltpu.sync_copy(x_hbm.at[pl.ds(wid * TPW, TPW)], idx_v.at[pl.ds(0, TPW)])
    scale = jnp.float32(1.0 / CTX)
    zeros = jnp.zeros((LANES,), jnp.float32)

    def elem(e, carry):
        descs = []
        for j in range(CTX):
            v = idx_v[pl.ds(e * CTX + j, LANES)][0]
            v8 = pl.multiple_of((v >> 3) * 8, 8)
            descs.append(
                pltpu.async_copy(emb_hbm.at[pl.ds(v8, 8), :], tiles_v.at[j], sem)
            )
        for d in descs:
            d.wait()
        accs = [zeros] * (DIM // LANES)
        for j in range(CTX):
            v = idx_v[pl.ds(e * CTX + j, LANES)][0]
            s = v & 7
            for k in range(DIM // LANES):
                accs[k] = accs[k] + tiles_v[j, s, pl.ds(k * LANES, LANES)]
        for k in range(DIM // LANES):
            h_v[e, pl.ds(k * LANES, LANES)] = accs[k] * scale
        for k in range(DIM // LANES, HP // LANES):
            h_v[e, pl.ds(k * LANES, LANES)] = zeros
        return carry

    lax.fori_loop(0, BPW, elem, 0)
    pltpu.sync_copy(h_v, h_hbm.at[pl.ds(wid * BPW, BPW)])


@jax.jit
def _pool(x_flat, emb):
    mesh = plsc.VectorSubcoreMesh(core_axis_name="c", subcore_axis_name="s")
    return pl.kernel(
        _pool_body,
        out_type=jax.ShapeDtypeStruct((BATCH, HP), jnp.float32),
        mesh=mesh,
        scratch_types=[
            pltpu.VMEM((TPW + LANES,), jnp.int32),
            pltpu.VMEM((CTX, 8, DIM), jnp.float32),
            pltpu.VMEM((BPW, HP), jnp.float32),
            pltpu.SemaphoreType.DMA,
        ],
    )(x_flat, emb)


VT = 2048


def _proj_body(h_ref, w_ref, o_ref):
    o_ref[...] = lax.dot_general(
        h_ref[:, :DIM], w_ref[...],
        dimension_numbers=(((1,), (1,)), ((), ())),
        preferred_element_type=jnp.float32,
    )


@jax.jit
def _project(h, W):
    grid = pl.cdiv(VOCAB, VT)
    return pl.pallas_call(
        _proj_body,
        grid=(grid,),
        in_specs=[
            pl.BlockSpec((BATCH, HP), lambda i: (0, 0)),
            pl.BlockSpec((VT, DIM), lambda i: (i, 0)),
        ],
        out_specs=pl.BlockSpec((BATCH, VT), lambda i: (0, i)),
        out_shape=jax.ShapeDtypeStruct((BATCH, VOCAB), jnp.float32),
        compiler_params=pltpu.CompilerParams(
            dimension_semantics=("parallel",),
        ),
    )(h, W)


def kernel(x, emb, W):
    x_flat = x.reshape(-1).astype(jnp.int32)
    h = _pool(x_flat, emb)
    return _project(h, W)


# speedup vs baseline: 2.7717x; 2.7717x over previous
import functools

import jax
import jax.numpy as jnp
from jax import lax
from jax.experimental import pallas as pl
from jax.experimental.pallas import tpu as pltpu
from jax.experimental.pallas import tpu_sc as plsc

VOCAB = 100000
DIM = 64
BATCH = 1024
CTX = 20

NUM_CORES = 2
NUM_SUBCORES = 16
LANES = 16
NW = NUM_CORES * NUM_SUBCORES
BPW = BATCH // NW
TPW = BPW * CTX
HP = 128
GCH = 128
NCH = TPW // GCH


def _pool_body(pair_hbm, off_hbm, emb2_hbm, h_hbm, idxp_v, offs_v, rows_v, h_v, sem):
    wid = lax.axis_index("s") * NUM_CORES + lax.axis_index("c")
    base = wid * TPW
    pltpu.sync_copy(pair_hbm.at[pl.ds(base, TPW)], idxp_v.at[pl.ds(0, TPW)])
    pltpu.sync_copy(off_hbm.at[pl.ds(base, TPW)], offs_v.at[pl.ds(0, TPW)])
    descs = []
    for c in range(NCH):
        descs.append(
            pltpu.async_copy(
                emb2_hbm.at[idxp_v.at[pl.ds(c * GCH, GCH)]],
                rows_v.at[pl.ds(c * GCH, GCH)],
                sem,
            )
        )
    for d in descs:
        d.wait()
    scale = jnp.float32(1.0 / CTX)
    zeros = jnp.zeros((LANES,), jnp.float32)

    def elem(e, carry):
        accs = [zeros] * (DIM // LANES)
        for j in range(CTX):
            t = e * CTX + j
            off = offs_v[pl.ds(t, LANES)][0]
            for k in range(DIM // LANES):
                accs[k] = accs[k] + rows_v[t, pl.ds(off + k * LANES, LANES)]
        for k in range(DIM // LANES):
            h_v[e, pl.ds(k * LANES, LANES)] = accs[k] * scale
        for k in range(DIM // LANES, HP // LANES):
            h_v[e, pl.ds(k * LANES, LANES)] = zeros
        return carry

    lax.fori_loop(0, BPW, elem, 0)
    pltpu.sync_copy(h_v, h_hbm.at[pl.ds(wid * BPW, BPW)])


@jax.jit
def _pool(pair_flat, off_flat, emb2):
    mesh = plsc.VectorSubcoreMesh(core_axis_name="c", subcore_axis_name="s")
    return pl.kernel(
        _pool_body,
        out_type=jax.ShapeDtypeStruct((BATCH, HP), jnp.float32),
        mesh=mesh,
        scratch_types=[
            pltpu.VMEM((TPW + LANES,), jnp.int32),
            pltpu.VMEM((TPW + LANES,), jnp.int32),
            pltpu.VMEM((TPW, HP), jnp.float32),
            pltpu.VMEM((BPW, HP), jnp.float32),
            pltpu.SemaphoreType.DMA,
        ],
    )(pair_flat, off_flat, emb2)


VT = 2048


def _proj_body(w_ref, h_ref, o_ref):
    o_ref[...] = lax.dot_general(
        w_ref[...], h_ref[:, :DIM],
        dimension_numbers=(((0,), (1,)), ((), ())),
        preferred_element_type=jnp.float32,
    )


@jax.jit
def _project(h, WT):
    grid = pl.cdiv(VOCAB, VT)
    return pl.pallas_call(
        _proj_body,
        grid=(grid,),
        in_specs=[
            pl.BlockSpec((DIM, VT), lambda i: (0, i)),
            pl.BlockSpec((BATCH, HP), lambda i: (0, 0)),
        ],
        out_specs=pl.BlockSpec((VT, BATCH), lambda i: (i, 0)),
        out_shape=jax.ShapeDtypeStruct((VOCAB, BATCH), jnp.float32),
        compiler_params=pltpu.CompilerParams(
            dimension_semantics=("parallel",),
        ),
    )(WT, h)


def kernel(x, emb, W):
    xi = x.reshape(-1).astype(jnp.int32)
    pair_flat = xi >> 1
    off_flat = (xi & 1) * DIM
    emb2 = emb.reshape(VOCAB // 2, 2 * DIM)
    h = _pool(pair_flat, off_flat, emb2)
    return _project(h, W.T).T


# speedup vs baseline: 2.8263x; 1.0197x over previous
import functools

import jax
import jax.numpy as jnp
from jax import lax
from jax.experimental import pallas as pl
from jax.experimental.pallas import tpu as pltpu
from jax.experimental.pallas import tpu_sc as plsc

VOCAB = 100000
DIM = 64
BATCH = 1024
CTX = 20

NUM_CORES = 2
NUM_SUBCORES = 16
LANES = 16
NW = NUM_CORES * NUM_SUBCORES
BPW = BATCH // NW
TPW = BPW * CTX
HP = 128
GCH = 128
NCH = TPW // GCH


def _pool_body(x_hbm, emb_hbm, h_hbm, idx_v, rows_v, h_v, sem):
    wid = lax.axis_index("s") * NUM_CORES + lax.axis_index("c")
    pltpu.sync_copy(x_hbm.at[pl.ds(wid * TPW, TPW)], idx_v)
    descs = []
    for c in range(NCH):
        descs.append(
            pltpu.async_copy(
                emb_hbm.at[idx_v.at[pl.ds(c * GCH, GCH)]],
                rows_v.at[pl.ds(c * GCH, GCH)],
                sem,
            )
        )
    for d in descs:
        d.wait()
    scale = jnp.float32(1.0 / CTX)
    zeros = jnp.zeros((LANES,), jnp.float32)

    def elem(e, carry):
        accs = [zeros] * (DIM // LANES)
        for j in range(CTX):
            t = e * CTX + j
            for k in range(DIM // LANES):
                accs[k] = accs[k] + rows_v[t, pl.ds(k * LANES, LANES)]
        for k in range(DIM // LANES):
            h_v[e, pl.ds(k * LANES, LANES)] = accs[k] * scale
        for k in range(DIM // LANES, HP // LANES):
            h_v[e, pl.ds(k * LANES, LANES)] = zeros
        return carry

    lax.fori_loop(0, BPW, elem, 0)
    pltpu.sync_copy(h_v, h_hbm.at[pl.ds(wid * BPW, BPW)])


@jax.jit
def _pool(x_flat, emb):
    mesh = plsc.VectorSubcoreMesh(core_axis_name="c", subcore_axis_name="s")
    return pl.kernel(
        _pool_body,
        out_type=jax.ShapeDtypeStruct((BATCH, HP), jnp.float32),
        mesh=mesh,
        scratch_types=[
            pltpu.VMEM((TPW,), jnp.int32),
            pltpu.VMEM((TPW, DIM), jnp.float32),
            pltpu.VMEM((BPW, HP), jnp.float32),
            pltpu.SemaphoreType.DMA,
        ],
        compiler_params=pltpu.CompilerParams(use_tc_tiling_on_sc=False),
    )(x_flat, emb)


VT = 2048


def _proj_body(w_ref, h_ref, o_ref):
    o_ref[...] = lax.dot_general(
        w_ref[...], h_ref[:, :DIM],
        dimension_numbers=(((0,), (1,)), ((), ())),
        preferred_element_type=jnp.float32,
    )


@jax.jit
def _project(h, WT):
    grid = pl.cdiv(VOCAB, VT)
    return pl.pallas_call(
        _proj_body,
        grid=(grid,),
        in_specs=[
            pl.BlockSpec((DIM, VT), lambda i: (0, i)),
            pl.BlockSpec((BATCH, HP), lambda i: (0, 0)),
        ],
        out_specs=pl.BlockSpec((VT, BATCH), lambda i: (i, 0)),
        out_shape=jax.ShapeDtypeStruct((VOCAB, BATCH), jnp.float32),
        compiler_params=pltpu.CompilerParams(
            dimension_semantics=("parallel",),
        ),
    )(WT, h)


def kernel(x, emb, W):
    x_flat = x.reshape(-1).astype(jnp.int32)
    h = _pool(x_flat, emb)
    return _project(h, W.T).T


# speedup vs baseline: 2.9502x; 1.0438x over previous
import functools

import jax
import jax.numpy as jnp
from jax import lax
from jax.experimental import pallas as pl
from jax.experimental.pallas import tpu as pltpu
from jax.experimental.pallas import tpu_sc as plsc

VOCAB = 100000
DIM = 64
BATCH = 1024
CTX = 20

NUM_CORES = 2
NUM_SUBCORES = 16
LANES = 16
NW = NUM_CORES * NUM_SUBCORES
BPW = BATCH // NW
TPW = BPW * CTX
HP = 128
GCH = 128
NCH = TPW // GCH

RB = 1024
RGRID = 49
HALF = RB * RGRID


def _repack_body(a_ref, b_ref, o_ref):
    o_ref[:, :DIM] = lax.transpose(a_ref[...], (1, 0))
    o_ref[:, DIM:] = lax.transpose(b_ref[...], (1, 0))


@jax.jit
def _repack(embT):
    return pl.pallas_call(
        _repack_body,
        grid=(RGRID,),
        in_specs=[
            pl.BlockSpec((DIM, RB), lambda i: (0, i)),
            pl.BlockSpec((DIM, RB), lambda i: (0, i + RGRID)),
        ],
        out_specs=pl.BlockSpec((RB, 2 * DIM), lambda i: (i, 0)),
        out_shape=jax.ShapeDtypeStruct((HALF, 2 * DIM), jnp.float32),
        compiler_params=pltpu.CompilerParams(
            dimension_semantics=("parallel",),
        ),
    )(embT, embT)


def _pool_body(pair_hbm, off_hbm, emb2_hbm, h_hbm, idxp_v, offs_v, rows_v, h_v, sem):
    wid = lax.axis_index("s") * NUM_CORES + lax.axis_index("c")
    base = wid * TPW
    pltpu.sync_copy(pair_hbm.at[pl.ds(base, TPW)], idxp_v.at[pl.ds(0, TPW)])
    pltpu.sync_copy(off_hbm.at[pl.ds(base, TPW)], offs_v.at[pl.ds(0, TPW)])
    descs = []
    for c in range(NCH):
        descs.append(
            pltpu.async_copy(
                emb2_hbm.at[idxp_v.at[pl.ds(c * GCH, GCH)]],
                rows_v.at[pl.ds(c * GCH, GCH)],
                sem,
            )
        )
    for d in descs:
        d.wait()
    scale = jnp.float32(1.0 / CTX)
    zeros = jnp.zeros((LANES,), jnp.float32)

    def elem(e, carry):
        accs = [zeros] * (DIM // LANES)
        for j in range(CTX):
            t = e * CTX + j
            off = offs_v[pl.ds(t, LANES)][0]
            for k in range(DIM // LANES):
                accs[k] = accs[k] + rows_v[t, pl.ds(off + k * LANES, LANES)]
        for k in range(DIM // LANES):
            h_v[e, pl.ds(k * LANES, LANES)] = accs[k] * scale
        for k in range(DIM // LANES, HP // LANES):
            h_v[e, pl.ds(k * LANES, LANES)] = zeros
        return carry

    lax.fori_loop(0, BPW, elem, 0)
    pltpu.sync_copy(h_v, h_hbm.at[pl.ds(wid * BPW, BPW)])


@jax.jit
def _pool(pair_flat, off_flat, emb2):
    mesh = plsc.VectorSubcoreMesh(core_axis_name="c", subcore_axis_name="s")
    return pl.kernel(
        _pool_body,
        out_type=jax.ShapeDtypeStruct((BATCH, HP), jnp.float32),
        mesh=mesh,
        scratch_types=[
            pltpu.VMEM((TPW + LANES,), jnp.int32),
            pltpu.VMEM((TPW + LANES,), jnp.int32),
            pltpu.VMEM((TPW, HP), jnp.float32),
            pltpu.VMEM((BPW, HP), jnp.float32),
            pltpu.SemaphoreType.DMA,
        ],
    )(pair_flat, off_flat, emb2)


VT = 2048


def _proj_body(w_ref, h_ref, o_ref):
    o_ref[...] = lax.dot_general(
        w_ref[...], h_ref[:, :DIM],
        dimension_numbers=(((0,), (1,)), ((), ())),
        preferred_element_type=jnp.float32,
    )


@jax.jit
def _project(h, WT):
    grid = pl.cdiv(VOCAB, VT)
    return pl.pallas_call(
        _proj_body,
        grid=(grid,),
        in_specs=[
            pl.BlockSpec((DIM, VT), lambda i: (0, i)),
            pl.BlockSpec((BATCH, HP), lambda i: (0, 0)),
        ],
        out_specs=pl.BlockSpec((VT, BATCH), lambda i: (i, 0)),
        out_shape=jax.ShapeDtypeStruct((VOCAB, BATCH), jnp.float32),
        compiler_params=pltpu.CompilerParams(
            dimension_semantics=("parallel",),
        ),
    )(WT, h)


def kernel(x, emb, W):
    xi = x.reshape(-1).astype(jnp.int32)
    hi = xi >= HALF
    pair_flat = jnp.where(hi, xi - HALF, xi)
    off_flat = jnp.where(hi, DIM, 0).astype(jnp.int32)
    emb2 = _repack(emb.T)
    h = _pool(pair_flat, off_flat, emb2)
    return _project(h, W.T).T
